# baseline (speedup 1.0000x reference)
import functools

import jax
import jax.numpy as jnp
from jax import lax
from jax.experimental import pallas as pl
from jax.experimental.pallas import tpu as pltpu
from jax.experimental.pallas import tpu_sc as plsc

N = 10000
E = 320000
D = 128
NC = 2
NS = 16
NW = NC * NS
CHUNK = 128
NCHUNK = E // CHUNK
NQ = NCHUNK // NW
NR = NCHUNK - NQ * NW
NG_MAX = NQ + 1
EPT = NQ * CHUNK
EXTRA0 = NW * EPT
N_PAD = 10240
ZROWS = N_PAD // NS

_MESH = plsc.VectorSubcoreMesh(core_axis_name="c", subcore_axis_name="s")


def _fill_f32(ref, n, value):
    vec = jnp.full((16,), value, jnp.float32)
    for j in range(n // 16):
        ref[pl.ds(j * 16, 16)] = vec


def _copy_chunk_idx(src_ref, dst_ref, chunk):
    for g in range(CHUNK // 16):
        dst_ref[pl.ds(g * 16, 16)] = src_ref[pl.ds(chunk * CHUNK + g * 16, 16)]


def _deg_body(dst_hbm, zeros_hbm, degp_hbm, deg_sh, didx_all, didx_c, ones_v):
    c = lax.axis_index("c")
    s = lax.axis_index("s")
    _fill_f32(ones_v, CHUNK, 1.0)
    off = pl.multiple_of(s * ZROWS, 8)
    pltpu.sync_copy(zeros_hbm, deg_sh.at[pl.ds(off, ZROWS)])
    plsc.subcore_barrier()

    w = c * NS + s
    base = pl.multiple_of(w * EPT, 8)
    pltpu.sync_copy(dst_hbm.at[pl.ds(base, EPT)], didx_all.at[pl.ds(0, EPT)])

    @pl.when(w < NR)
    def _():
        pltpu.sync_copy(dst_hbm.at[pl.ds(pl.multiple_of(EXTRA0 + w * CHUNK, 8),
                                         CHUNK)],
                        didx_all.at[pl.ds(EPT, CHUNK)])

    for i in range(NQ):
        _copy_chunk_idx(didx_all, didx_c, i)
        pltpu.sync_copy(ones_v, deg_sh.at[didx_c], add=True)

    @pl.when(w < NR)
    def _():
        _copy_chunk_idx(didx_all, didx_c, NQ)
        pltpu.sync_copy(ones_v, deg_sh.at[didx_c], add=True)

    plsc.subcore_barrier()
    pltpu.sync_copy(deg_sh.at[pl.ds(off, ZROWS)],
                    degp_hbm.at[c, pl.ds(off, ZROWS)])


def _scat_body(src_hbm, dst_hbm, xs_hbm, zeros_hbm, part_hbm,
               acc_sh, sidx_all, didx_v, rows_v, gsem, dsem):
    c = lax.axis_index("c")
    s = lax.axis_index("s")
    off = pl.multiple_of(s * ZROWS, 8)
    pltpu.sync_copy(zeros_hbm, acc_sh.at[pl.ds(off, ZROWS)])
    plsc.subcore_barrier()

    w = c * NS + s
    ng = NQ + jnp.where(w < NR, 1, 0)
    base = pl.multiple_of(w * EPT, 128)
    pltpu.sync_copy(src_hbm.at[pl.ds(base, EPT)], sidx_all.at[pl.ds(0, EPT)])

    @pl.when(w < NR)
    def _():
        pltpu.sync_copy(src_hbm.at[pl.ds(pl.multiple_of(EXTRA0 + w * CHUNK, 8),
                                         CHUNK)],
                        sidx_all.at[pl.ds(EPT, CHUNK)])

    def gather_args(i, p):
        off = pl.multiple_of(i * CHUNK, 128)
        return (xs_hbm.at[sidx_all.at[pl.ds(off, CHUNK)]],
                rows_v.at[p], gsem.at[p])

    def didx_args(i, p):
        off = pl.multiple_of(
            jnp.where(i == NQ, EXTRA0 + w * CHUNK, base + i * CHUNK), 128)
        return dst_hbm.at[pl.ds(off, CHUNK)], didx_v.at[p, 0], dsem.at[p]

    pltpu.async_copy(*gather_args(0, 0))
    pltpu.async_copy(*didx_args(0, 0))

    def body(i, carry):
        p = lax.rem(i, 2)
        pn = 1 - p

        @pl.when(i + 1 < ng)
        def _():
            pltpu.async_copy(*gather_args(i + 1, pn))
            pltpu.async_copy(*didx_args(i + 1, pn))

        pltpu.make_async_copy(*gather_args(i, p)).wait()
        pltpu.make_async_copy(*didx_args(i, p)).wait()
        pltpu.sync_copy(rows_v.at[p], acc_sh.at[didx_v.at[p, 0]], add=True)
        return carry

    lax.fori_loop(0, ng, body, 0)

    plsc.subcore_barrier()
    pltpu.sync_copy(acc_sh.at[pl.ds(off, ZROWS)],
                    part_hbm.at[c, pl.ds(off, ZROWS)])


def _dis_col(degp):
    deg = degp[0:1, :] + degp[1:2, :]
    dis = jnp.where(deg > 0, lax.rsqrt(jnp.maximum(deg, 1e-12)), 0.0)
    return jnp.reshape(dis, (N_PAD, 1))[:N]


def _scale_body(x_ref, degp_ref, xs_ref):
    xs_ref[...] = x_ref[...] * _dis_col(degp_ref[...])


def _comb_body(part_ref, degp_ref, out_ref):
    out_ref[...] = ((part_ref[0, :N] + part_ref[1, :N])
                    * _dis_col(degp_ref[...]))


_deg_kernel = pl.kernel(
    _deg_body,
    out_type=jax.ShapeDtypeStruct((NC, N_PAD), jnp.float32),
    mesh=_MESH,
    scratch_types=[
        pltpu.VMEM_SHARED((N_PAD,), jnp.float32),
        pltpu.VMEM((NG_MAX * CHUNK,), jnp.int32),
        pltpu.VMEM((CHUNK,), jnp.int32),
        pltpu.VMEM((CHUNK,), jnp.float32),
    ],
)

_scat_kernel = pl.kernel(
    _scat_body,
    out_type=jax.ShapeDtypeStruct((NC, N_PAD, D), jnp.float32),
    mesh=_MESH,
    scratch_types=[
        pltpu.VMEM_SHARED((N_PAD, D), jnp.float32),
        pltpu.VMEM((NG_MAX * CHUNK,), jnp.int32),
        pltpu.VMEM((2, 8, CHUNK), jnp.int32),
        pltpu.VMEM((2, CHUNK, D), jnp.float32),
        pltpu.SemaphoreType.DMA((2,)),
        pltpu.SemaphoreType.DMA((2,)),
    ],
)

_scale_call = pl.pallas_call(
    _scale_body, out_shape=jax.ShapeDtypeStruct((N, D), jnp.float32))

_comb_call = pl.pallas_call(
    _comb_body, out_shape=jax.ShapeDtypeStruct((N, D), jnp.float32))


def kernel(x, edge_index):
    ei = edge_index.astype(jnp.int32)
    src = ei[0]
    dst = ei[1]
    zeros = jnp.zeros((ZROWS, D), jnp.float32)

    degp = _deg_kernel(dst, jnp.zeros((ZROWS,), jnp.float32))
    xs = _scale_call(x, degp)
    part = _scat_kernel(src, dst, xs, zeros)
    return _comb_call(part, degp)


# speedup vs baseline: 1.0865x; 1.0865x over previous
import functools

import jax
import jax.numpy as jnp
from jax import lax
from jax.experimental import pallas as pl
from jax.experimental.pallas import tpu as pltpu
from jax.experimental.pallas import tpu_sc as plsc

N = 10000
E = 320000
D = 128
NC = 2
NS = 16
NW = NC * NS
CHUNK = 128
NCHUNK = E // CHUNK
NQ = NCHUNK // NW
NR = NCHUNK - NQ * NW
NG_MAX = NQ + 1
EPT = NQ * CHUNK
EXTRA0 = NW * EPT
N_PAD = 10240
ZROWS = N_PAD // NS

_MESH = plsc.VectorSubcoreMesh(core_axis_name="c", subcore_axis_name="s")


def _fill_f32(ref, n, value):
    vec = jnp.full((16,), value, jnp.float32)
    for j in range(n // 16):
        ref[pl.ds(j * 16, 16)] = vec


def _copy_chunk_idx(src_ref, dst_ref, chunk):
    for g in range(CHUNK // 16):
        dst_ref[pl.ds(g * 16, 16)] = src_ref[pl.ds(chunk * CHUNK + g * 16, 16)]


def _deg_body(ei_hbm, zeros_hbm, degp_hbm, deg_sh, didx_all, didx_c, ones_v):
    c = lax.axis_index("c")
    s = lax.axis_index("s")
    _fill_f32(ones_v, CHUNK, 1.0)
    off = pl.multiple_of(s * ZROWS, 8)
    pltpu.sync_copy(zeros_hbm, deg_sh.at[pl.ds(off, ZROWS)])
    plsc.subcore_barrier()

    w = c * NS + s
    base = pl.multiple_of(w * EPT, 8)
    pltpu.sync_copy(ei_hbm.at[1, pl.ds(base, EPT)], didx_all.at[pl.ds(0, EPT)])

    @pl.when(w < NR)
    def _():
        pltpu.sync_copy(ei_hbm.at[1, pl.ds(pl.multiple_of(EXTRA0 + w * CHUNK, 8),
                                            CHUNK)],
                        didx_all.at[pl.ds(EPT, CHUNK)])

    for i in range(NQ):
        _copy_chunk_idx(didx_all, didx_c, i)
        pltpu.sync_copy(ones_v, deg_sh.at[didx_c], add=True)

    @pl.when(w < NR)
    def _():
        _copy_chunk_idx(didx_all, didx_c, NQ)
        pltpu.sync_copy(ones_v, deg_sh.at[didx_c], add=True)

    plsc.subcore_barrier()
    pltpu.sync_copy(deg_sh.at[pl.ds(off, ZROWS)],
                    degp_hbm.at[c, pl.ds(off, ZROWS)])


def _scat_body(ei_hbm, xs_hbm, zeros_hbm, part_hbm,
               acc_sh, sidx_all, didx_v, rows_v, gsem, dsem):
    c = lax.axis_index("c")
    s = lax.axis_index("s")
    off = pl.multiple_of(s * ZROWS, 8)
    pltpu.sync_copy(zeros_hbm, acc_sh.at[pl.ds(off, ZROWS)])
    plsc.subcore_barrier()

    w = c * NS + s
    ng = NQ + jnp.where(w < NR, 1, 0)
    base = pl.multiple_of(w * EPT, 128)
    pltpu.sync_copy(ei_hbm.at[0, pl.ds(base, EPT)], sidx_all.at[pl.ds(0, EPT)])

    @pl.when(w < NR)
    def _():
        pltpu.sync_copy(ei_hbm.at[0, pl.ds(pl.multiple_of(EXTRA0 + w * CHUNK, 8),
                                            CHUNK)],
                        sidx_all.at[pl.ds(EPT, CHUNK)])

    def gather_args(i, p):
        off = pl.multiple_of(i * CHUNK, 128)
        return (xs_hbm.at[sidx_all.at[pl.ds(off, CHUNK)]],
                rows_v.at[p], gsem.at[p])

    def didx_args(i, p):
        off = pl.multiple_of(
            jnp.where(i == NQ, EXTRA0 + w * CHUNK, base + i * CHUNK), 128)
        return ei_hbm.at[1, pl.ds(off, CHUNK)], didx_v.at[p, 0], dsem.at[p]

    pltpu.async_copy(*gather_args(0, 0))
    pltpu.async_copy(*didx_args(0, 0))

    def body(i, carry):
        p = lax.rem(i, 2)
        pn = 1 - p

        @pl.when(i + 1 < ng)
        def _():
            pltpu.async_copy(*gather_args(i + 1, pn))
            pltpu.async_copy(*didx_args(i + 1, pn))

        pltpu.make_async_copy(*gather_args(i, p)).wait()
        pltpu.make_async_copy(*didx_args(i, p)).wait()
        pltpu.sync_copy(rows_v.at[p], acc_sh.at[didx_v.at[p, 0]], add=True)
        return carry

    lax.fori_loop(0, ng, body, 0)

    plsc.subcore_barrier()
    pltpu.sync_copy(acc_sh.at[pl.ds(off, ZROWS)],
                    part_hbm.at[c, pl.ds(off, ZROWS)])


def _dis_col(degp):
    deg = degp[0:1, :] + degp[1:2, :]
    dis = jnp.where(deg > 0, lax.rsqrt(jnp.maximum(deg, 1e-12)), 0.0)
    return jnp.reshape(dis, (N_PAD, 1))[:N]


def _scale_body(x_ref, degp_ref, xs_ref):
    xs_ref[...] = x_ref[...] * _dis_col(degp_ref[...])


def _comb_body(part_ref, degp_ref, out_ref):
    out_ref[...] = ((part_ref[0, :N] + part_ref[1, :N])
                    * _dis_col(degp_ref[...]))


_deg_kernel = pl.kernel(
    _deg_body,
    out_type=jax.ShapeDtypeStruct((NC, N_PAD), jnp.float32),
    mesh=_MESH,
    scratch_types=[
        pltpu.VMEM_SHARED((N_PAD,), jnp.float32),
        pltpu.VMEM((NG_MAX * CHUNK,), jnp.int32),
        pltpu.VMEM((CHUNK,), jnp.int32),
        pltpu.VMEM((CHUNK,), jnp.float32),
    ],
)

_scat_kernel = pl.kernel(
    _scat_body,
    out_type=jax.ShapeDtypeStruct((NC, N_PAD, D), jnp.float32),
    mesh=_MESH,
    scratch_types=[
        pltpu.VMEM_SHARED((N_PAD, D), jnp.float32),
        pltpu.VMEM((NG_MAX * CHUNK,), jnp.int32),
        pltpu.VMEM((2, 8, CHUNK), jnp.int32),
        pltpu.VMEM((2, CHUNK, D), jnp.float32),
        pltpu.SemaphoreType.DMA((2,)),
        pltpu.SemaphoreType.DMA((2,)),
    ],
)

_scale_call = pl.pallas_call(
    _scale_body, out_shape=jax.ShapeDtypeStruct((N, D), jnp.float32))

_comb_call = pl.pallas_call(
    _comb_body, out_shape=jax.ShapeDtypeStruct((N, D), jnp.float32))


def kernel(x, edge_index):
    ei = edge_index.astype(jnp.int32)
    zeros = jnp.zeros((ZROWS, D), jnp.float32)

    degp = _deg_kernel(ei, jnp.zeros((ZROWS,), jnp.float32))
    xs = _scale_call(x, degp)
    part = _scat_kernel(ei, xs, zeros)
    return _comb_call(part, degp)


# speedup vs baseline: 1.1129x; 1.0243x over previous
import functools

import jax
import jax.numpy as jnp
from jax import lax
from jax.experimental import pallas as pl
from jax.experimental.pallas import tpu as pltpu
from jax.experimental.pallas import tpu_sc as plsc

N = 10000
E = 320000
D = 128
NC = 2
NS = 16
NW = NC * NS
CHUNK = 128
NCHUNK = E // CHUNK
NQ = NCHUNK // NW
NR = NCHUNK - NQ * NW
NG_MAX = NQ + 1
EPT = NQ * CHUNK
EXTRA0 = NW * EPT
N_PAD = 10240
ZROWS = N_PAD // NS

_MESH = plsc.VectorSubcoreMesh(core_axis_name="c", subcore_axis_name="s")


def _fill_f32(ref, n, value):
    vec = jnp.full((16,), value, jnp.float32)
    for j in range(n // 16):
        ref[pl.ds(j * 16, 16)] = vec


def _copy_chunk_idx(src_ref, dst_ref, chunk):
    for g in range(CHUNK // 16):
        dst_ref[pl.ds(g * 16, 16)] = src_ref[pl.ds(chunk * CHUNK + g * 16, 16)]


def _deg_body(ei_hbm, zeros_hbm, degp_hbm, deg_sh, didx_all, didx_ca, didx_cb,
              ones_v, ssem_a, ssem_b):
    c = lax.axis_index("c")
    s = lax.axis_index("s")
    _fill_f32(ones_v, CHUNK, 1.0)
    off = pl.multiple_of(s * ZROWS, 8)
    pltpu.sync_copy(zeros_hbm, deg_sh.at[pl.ds(off, ZROWS)])
    plsc.subcore_barrier()

    w = c * NS + s
    base = pl.multiple_of(w * EPT, 8)
    pltpu.sync_copy(ei_hbm.at[1, pl.ds(base, EPT)], didx_all.at[pl.ds(0, EPT)])

    @pl.when(w < NR)
    def _():
        pltpu.sync_copy(ei_hbm.at[1, pl.ds(pl.multiple_of(EXTRA0 + w * CHUNK, 8),
                                            CHUNK)],
                        didx_all.at[pl.ds(EPT, CHUNK)])

    def scat_args(i):
        buf = didx_ca if i % 2 == 0 else didx_cb
        sem = ssem_a if i % 2 == 0 else ssem_b
        return buf, sem

    for i in range(NQ):
        buf, sem = scat_args(i)
        if i >= 2:
            pltpu.make_async_copy(ones_v, deg_sh.at[buf], sem).wait()
        _copy_chunk_idx(didx_all, buf, i)
        pltpu.async_copy(ones_v, deg_sh.at[buf], sem, add=True)

    @pl.when(w < NR)
    def _():
        buf, sem = scat_args(NQ)
        pltpu.make_async_copy(ones_v, deg_sh.at[buf], sem).wait()
        _copy_chunk_idx(didx_all, buf, NQ)
        pltpu.async_copy(ones_v, deg_sh.at[buf], sem, add=True)

    pltpu.make_async_copy(ones_v, deg_sh.at[didx_ca], ssem_a).wait()
    pltpu.make_async_copy(ones_v, deg_sh.at[didx_cb], ssem_b).wait()
    plsc.subcore_barrier()
    pltpu.sync_copy(deg_sh.at[pl.ds(off, ZROWS)],
                    degp_hbm.at[c, pl.ds(off, ZROWS)])


def _scat_body(ei_hbm, xs_hbm, zeros_hbm, part_hbm,
               acc_sh, sidx_all, didx_v, rows_v, gsem, dsem):
    c = lax.axis_index("c")
    s = lax.axis_index("s")
    off = pl.multiple_of(s * ZROWS, 8)
    pltpu.sync_copy(zeros_hbm, acc_sh.at[pl.ds(off, ZROWS)])
    plsc.subcore_barrier()

    w = c * NS + s
    ng = NQ + jnp.where(w < NR, 1, 0)
    base = pl.multiple_of(w * EPT, 128)
    pltpu.sync_copy(ei_hbm.at[0, pl.ds(base, EPT)], sidx_all.at[pl.ds(0, EPT)])

    @pl.when(w < NR)
    def _():
        pltpu.sync_copy(ei_hbm.at[0, pl.ds(pl.multiple_of(EXTRA0 + w * CHUNK, 8),
                                            CHUNK)],
                        sidx_all.at[pl.ds(EPT, CHUNK)])

    def gather_args(i, p):
        off = pl.multiple_of(i * CHUNK, 128)
        return (xs_hbm.at[sidx_all.at[pl.ds(off, CHUNK)]],
                rows_v.at[p], gsem.at[p])

    def didx_args(i, p):
        off = pl.multiple_of(
            jnp.where(i == NQ, EXTRA0 + w * CHUNK, base + i * CHUNK), 128)
        return ei_hbm.at[1, pl.ds(off, CHUNK)], didx_v.at[p, 0], dsem.at[p]

    pltpu.async_copy(*gather_args(0, 0))
    pltpu.async_copy(*didx_args(0, 0))

    def body(i, carry):
        p = lax.rem(i, 2)
        pn = 1 - p

        @pl.when(i + 1 < ng)
        def _():
            pltpu.async_copy(*gather_args(i + 1, pn))
            pltpu.async_copy(*didx_args(i + 1, pn))

        pltpu.make_async_copy(*gather_args(i, p)).wait()
        pltpu.make_async_copy(*didx_args(i, p)).wait()
        pltpu.sync_copy(rows_v.at[p], acc_sh.at[didx_v.at[p, 0]], add=True)
        return carry

    lax.fori_loop(0, ng, body, 0)

    plsc.subcore_barrier()
    pltpu.sync_copy(acc_sh.at[pl.ds(off, ZROWS)],
                    part_hbm.at[c, pl.ds(off, ZROWS)])


def _dis_col(degp):
    deg = degp[0:1, :] + degp[1:2, :]
    dis = jnp.where(deg > 0, lax.rsqrt(jnp.maximum(deg, 1e-12)), 0.0)
    return jnp.reshape(dis, (N_PAD, 1))[:N]


def _scale_body(x_ref, degp_ref, xs_ref):
    xs_ref[...] = x_ref[...] * _dis_col(degp_ref[...])


def _comb_body(part_ref, degp_ref, out_ref):
    out_ref[...] = ((part_ref[0, :N] + part_ref[1, :N])
                    * _dis_col(degp_ref[...]))


_deg_kernel = pl.kernel(
    _deg_body,
    out_type=jax.ShapeDtypeStruct((NC, N_PAD), jnp.float32),
    mesh=_MESH,
    scratch_types=[
        pltpu.VMEM_SHARED((N_PAD,), jnp.float32),
        pltpu.VMEM((NG_MAX * CHUNK,), jnp.int32),
        pltpu.VMEM((CHUNK,), jnp.int32),
        pltpu.VMEM((CHUNK,), jnp.int32),
        pltpu.VMEM((CHUNK,), jnp.float32),
        pltpu.SemaphoreType.DMA,
        pltpu.SemaphoreType.DMA,
    ],
)

_scat_kernel = pl.kernel(
    _scat_body,
    out_type=jax.ShapeDtypeStruct((NC, N_PAD, D), jnp.float32),
    mesh=_MESH,
    scratch_types=[
        pltpu.VMEM_SHARED((N_PAD, D), jnp.float32),
        pltpu.VMEM((NG_MAX * CHUNK,), jnp.int32),
        pltpu.VMEM((2, 8, CHUNK), jnp.int32),
        pltpu.VMEM((2, CHUNK, D), jnp.float32),
        pltpu.SemaphoreType.DMA((2,)),
        pltpu.SemaphoreType.DMA((2,)),
    ],
)

_scale_call = pl.pallas_call(
    _scale_body, out_shape=jax.ShapeDtypeStruct((N, D), jnp.float32))

_comb_call = pl.pallas_call(
    _comb_body, out_shape=jax.ShapeDtypeStruct((N, D), jnp.float32))


def kernel(x, edge_index):
    ei = edge_index.astype(jnp.int32)
    zeros = jnp.zeros((ZROWS, D), jnp.float32)

    degp = _deg_kernel(ei, jnp.zeros((ZROWS,), jnp.float32))
    xs = _scale_call(x, degp)
    part = _scat_kernel(ei, xs, zeros)
    return _comb_call(part, degp)
